# baseline (speedup 1.0000x reference)
import functools

import jax
import jax.numpy as jnp
import numpy as np
from jax import lax
from jax.experimental import pallas as pl
from jax.experimental.pallas import tpu as pltpu
from jax.experimental.pallas import tpu_sc as plsc

N = 10000
E = 320000
H = 128
RV = 10
SH = H - RV
L = 3

NC = 2
NS = 16
NW = NC * NS
CH = 64
NG = CH // 16
NCHUNK = 158
EW = NCHUNK * CH
EP = NW * EW
NCHT = NW * NCHUNK
NP2 = 10112
RPT = NP2 // NS
CW = SH // 2
RW = RV // 2
EREC = 19 * CH

_sc_mesh = plsc.VectorSubcoreMesh(core_axis_name="c", subcore_axis_name="s")


def _unpack2(w):
    lo = plsc.bitcast(jnp.left_shift(w, 16), jnp.float32)
    hi = plsc.bitcast(jnp.bitwise_and(w, -65536), jnp.float32)
    return lo, hi


def _compute_chunk(eb, hb, combo_v):
    @plsc.parallel_loop(0, CH // 8)
    def _(bi):
        for i in range(8):
            e = bi * 8 + i
            rv = eb[pl.ds(pl.multiple_of(3 * CH + e * 16, 16), 16)]
            rlo, rhi = _unpack2(rv)
            for k in range(4):
                hw = plsc.bitcast(hb[e, pl.ds(16 * k, 16)], jnp.int32)
                cw = combo_v[e, pl.ds(16 * k, 16)]
                hlo, hhi = _unpack2(hw)
                clo, chi = _unpack2(cw)
                if k == 3:
                    hlo = hlo + rlo
                    hhi = hhi + rhi
                hb[e, pl.ds(16 * k, 16)] = jnp.maximum(hlo + clo, 0.0)
                hb[e, pl.ds(64 + 16 * k, 16)] = jnp.maximum(hhi + chi, 0.0)


def _msg_body(h, edata, comboP, zrows, out,
              shared, scombo, ebufA, ebufB, dbuf, hbufA, hbufB, cbufA, cbufB,
              gsemA, gsemB, lsemA, lsemB, csemA, csemB, sem_s):
    c = lax.axis_index("c")
    s = lax.axis_index("s")

    @pl.when(s == 0)
    def _():
        pltpu.sync_copy(comboP, scombo)

    pltpu.sync_copy(zrows, shared.at[pl.ds(s * RPT, RPT)])
    plsc.subcore_barrier()

    ci0 = (c * NS + s) * NCHUNK
    ebufs = (ebufA, ebufB)
    hbufs = (hbufA, hbufB)
    cbufs = (cbufA, cbufB)
    gsems = (gsemA, gsemB)
    lsems = (lsemA, lsemB)
    csems = (csemA, csemB)

    def src_idx(eb):
        return eb.at[pl.ds(0, CH)]

    def cid_idx(eb):
        return eb.at[pl.ds(2 * CH, CH)]

    pltpu.sync_copy(edata.at[pl.ds(ci0 * EREC, EREC)], ebufA)
    pltpu.async_copy(edata.at[pl.ds((ci0 + 1) * EREC, EREC)], ebufB, lsemB)
    pltpu.async_copy(h.at[src_idx(ebufA)], hbufA, gsemA)
    pltpu.async_copy(scombo.at[cid_idx(ebufA)], cbufA, csemA)

    def pair_body(p, carry):
        for b in range(2):
            eb, hb, cb = ebufs[b], hbufs[b], cbufs[b]
            ebn, hbn, cbn = ebufs[1 - b], hbufs[1 - b], cbufs[1 - b]
            j = p * 2 + b

            def wait_scatter(hbn=hbn, b=b):
                pltpu.make_async_copy(
                    hbn, shared.at[dbuf.at[1 - b]], sem_s).wait()
            if b == 0:
                @pl.when(p > 0)
                def _(wait_scatter=wait_scatter):
                    wait_scatter()
            else:
                wait_scatter()
            pltpu.make_async_copy(edata.at[pl.ds(0, EREC)], ebn,
                                  lsems[1 - b]).wait()
            pltpu.async_copy(h.at[src_idx(ebn)], hbn, gsems[1 - b])
            pltpu.async_copy(scombo.at[cid_idx(ebn)], cbn, csems[1 - b])
            pltpu.make_async_copy(h.at[src_idx(eb)], hb, gsems[b]).wait()
            pltpu.make_async_copy(scombo.at[cid_idx(eb)], cb, csems[b]).wait()
            for k in range(NG):
                dbuf[b, pl.ds(k * 16, 16)] = eb[pl.ds(CH + k * 16, 16)]
            _compute_chunk(eb, hb, cb)
            ci2 = ci0 + jnp.minimum(j + 2, NCHUNK - 1)
            pltpu.async_copy(edata.at[pl.ds(ci2 * EREC, EREC)], eb, lsems[b])
            pltpu.async_copy(hb, shared.at[dbuf.at[b]], sem_s, add=True)
        return carry

    lax.fori_loop(0, NCHUNK // 2, pair_body, 0)
    pltpu.make_async_copy(hbufB, shared.at[dbuf.at[1]], sem_s).wait()
    pltpu.make_async_copy(h.at[src_idx(ebufA)], hbufA, gsemA).wait()
    pltpu.make_async_copy(scombo.at[cid_idx(ebufA)], cbufA, csemA).wait()
    pltpu.make_async_copy(edata.at[pl.ds(0, EREC)], ebufB, lsemB).wait()
    plsc.subcore_barrier()

    rb = s * RPT
    pltpu.sync_copy(shared.at[pl.ds(rb, RPT)], out.at[c, pl.ds(rb, RPT)])


_msg_kernel = functools.partial(
    pl.kernel,
    mesh=_sc_mesh,
    compiler_params=pltpu.CompilerParams(needs_layout_passes=False),
    out_type=jax.ShapeDtypeStruct((NC, NP2, H), jnp.float32),
    scratch_types=[
        pltpu.VMEM_SHARED((NP2, H), jnp.float32),
        pltpu.VMEM_SHARED((512, 128), jnp.int32),
        pltpu.VMEM((EREC,), jnp.int32),
        pltpu.VMEM((EREC,), jnp.int32),
        pltpu.VMEM((2, CH), jnp.int32),
        pltpu.VMEM((CH, H), jnp.float32),
        pltpu.VMEM((CH, H), jnp.float32),
        pltpu.VMEM((CH, 128), jnp.int32),
        pltpu.VMEM((CH, 128), jnp.int32),
        pltpu.SemaphoreType.DMA,
        pltpu.SemaphoreType.DMA,
        pltpu.SemaphoreType.DMA,
        pltpu.SemaphoreType.DMA,
        pltpu.SemaphoreType.DMA,
        pltpu.SemaphoreType.DMA,
        pltpu.SemaphoreType.DMA,
    ],
)(_msg_body)


def _tc_pack(v):
    b = lax.bitcast_convert_type(v.astype(jnp.bfloat16), jnp.uint16)
    lo = b[:, :64].astype(jnp.uint32)
    hi = b[:, 64:].astype(jnp.uint32)
    p = lax.bitcast_convert_type(lo | (hi << 16), jnp.float32)
    return jnp.concatenate([p, p], axis=1)


def _enc_body(nf_ref, emb_ref, rx_ref, out_ref, pk_ref):
    acc = rx_ref[...]
    nf = nf_ref[...]
    for i in range(9):
        col = nf[:, i][:, None]
        oh = (col == lax.broadcasted_iota(jnp.int32, (1, 64), 1)).astype(jnp.float32)
        acc = acc + jnp.dot(oh, emb_ref[i], preferred_element_type=jnp.float32)
    out_ref[...] = acc
    pk_ref[...] = _tc_pack(acc)


def _mlp_body(h_ref, a0_ref, a1_ref, w1_ref, b1_ref, g1_ref, be1_ref,
              w2_ref, b2_ref, go_ref, bo_ref, eps_ref, out_ref, pk_ref, *,
              relu_out):
    x = h_ref[...]
    t = (1.0 + eps_ref[0, 0]) * x + a0_ref[0] + a1_ref[0]
    u = jnp.dot(t, w1_ref[...], preferred_element_type=jnp.float32) + b1_ref[...]
    u = jnp.maximum(g1_ref[...] * u + be1_ref[...], 0.0)
    v = jnp.dot(u, w2_ref[...], preferred_element_type=jnp.float32) + b2_ref[...]
    v = go_ref[...] * v + bo_ref[...]
    if relu_out:
        v = jnp.maximum(v, 0.0)
    out_ref[...] = v
    pk_ref[...] = _tc_pack(v)


_R = 400
_G = N // _R

_row_spec = pl.BlockSpec((_R, H), lambda i: (i, 0))
_pk_spec = pl.BlockSpec((_R, H), lambda i: (i, 0))
_vec_spec = pl.BlockSpec((1, H), lambda i: (0, 0))
_mat_spec = pl.BlockSpec((H, H), lambda i: (0, 0))
_out_shapes = [jax.ShapeDtypeStruct((N, H), jnp.float32),
               jax.ShapeDtypeStruct((N, H), jnp.float32)]


def _make_mlp(relu_out):
    return pl.pallas_call(
        functools.partial(_mlp_body, relu_out=relu_out),
        grid=(_G,),
        in_specs=[_row_spec,
                  pl.BlockSpec((1, _R, H), lambda i: (0, i, 0)),
                  pl.BlockSpec((1, _R, H), lambda i: (1, i, 0)),
                  _mat_spec, _vec_spec,
                  _vec_spec, _vec_spec, _mat_spec, _vec_spec, _vec_spec,
                  _vec_spec, pl.BlockSpec((1, 1), lambda i: (0, 0))],
        out_specs=[_row_spec, _pk_spec],
        out_shape=_out_shapes,
    )


_mlp_relu = _make_mlp(True)
_mlp_last = _make_mlp(False)

_encoder = pl.pallas_call(
    _enc_body,
    grid=(_G,),
    in_specs=[pl.BlockSpec((_R, 128), lambda i: (i, 0)),
              pl.BlockSpec((9, 64, H), lambda i: (0, 0, 0)),
              _row_spec],
    out_specs=[_row_spec, _pk_spec],
    out_shape=_out_shapes,
)


def _pack_split(x, k):
    bits = lax.bitcast_convert_type(x.astype(jnp.bfloat16), jnp.uint16)
    bits = bits.astype(jnp.uint32)
    return (bits[..., :k] | (bits[..., k:2 * k] << 16)).astype(jnp.int32)


_PERM = np.concatenate([np.arange(0, 59), np.arange(118, 123),
                        np.arange(59, 118), np.arange(123, 128)])


def kernel(rand_x, rand_edge, edge_index, node_feat, edge_attr, atom_emb,
           bond_emb, W1, b1, g1, be1, W2, b2, go, bo, eps):
    f32 = jnp.float32
    src = edge_index[0]
    dst = edge_index[1]
    cid = edge_attr[:, 0] * 64 + edge_attr[:, 1] * 8 + edge_attr[:, 2]
    pad = EP - E
    srcp = jnp.concatenate([src, jnp.zeros((pad,), jnp.int32)])
    dstp = jnp.concatenate([dst, jnp.full((pad,), N, jnp.int32)])
    cidp = jnp.concatenate([cid, jnp.zeros((pad,), jnp.int32)])
    randp = jnp.concatenate([_pack_split(rand_edge, RW),
                             jnp.zeros((pad, RW), jnp.int32)], axis=0)
    randp = jnp.concatenate([jnp.zeros((EP, 16 - RW), jnp.int32), randp],
                            axis=1)
    base3 = jnp.stack([srcp, dstp, cidp]).reshape(3, NCHT, CH)
    base3 = base3.transpose(1, 0, 2).reshape(NCHT, 3 * CH)
    randb = randp.reshape(NCHT, CH * 16)
    edata = jnp.concatenate([base3, randb], axis=1).reshape(-1)

    ii = jnp.arange(512)
    combo = (bond_emb[:, 0, ii // 64, :] + bond_emb[:, 1, (ii // 8) % 8, :]
             + bond_emb[:, 2, ii % 8, :])
    comboP = jnp.concatenate([_pack_split(combo, CW),
                              jnp.zeros((L, 512, 128 - CW), jnp.int32)],
                             axis=2)
    zrows = jnp.zeros((RPT, H), f32)

    nz = jnp.zeros((N, 5), f32)
    ez = jnp.zeros((9, 64, 5), f32)
    nfp = jnp.pad(node_feat, ((0, 0), (0, 128 - 9)))
    rxp = jnp.concatenate([jnp.zeros((N, CW), f32), rand_x[:, :5],
                           jnp.zeros((N, CW), f32), rand_x[:, 5:]], axis=1)
    embp = jnp.concatenate([atom_emb[:, :, :CW], ez,
                            atom_emb[:, :, CW:SH], ez], axis=2)

    h, hp = _encoder(nfp, embp, rxp)
    for l in range(L):
        sc_out = _msg_kernel(hp, edata, comboP[l], zrows)
        last = l == L - 1
        mlp = _mlp_last if last else _mlp_relu
        w2l = W2[l] if last else W2[l][:, _PERM]
        b2l = b2[l] if last else b2[l][_PERM]
        gol = go[l] if last else go[l][_PERM]
        bol = bo[l] if last else bo[l][_PERM]
        h, hp = mlp(h, sc_out, sc_out, W1[l][_PERM, :], b1[l][None],
                    g1[l][None], be1[l][None], w2l, b2l[None], gol[None],
                    bol[None], eps[l].reshape(1, 1))
    return h


# speedup vs baseline: 1.0808x; 1.0808x over previous
import functools

import jax
import jax.numpy as jnp
import numpy as np
from jax import lax
from jax.experimental import pallas as pl
from jax.experimental.pallas import tpu as pltpu
from jax.experimental.pallas import tpu_sc as plsc

N = 10000
E = 320000
H = 128
RV = 10
SH = H - RV
L = 3

NC = 2
NS = 16
NW = NC * NS
CH = 64
NG = CH // 16
NCK0 = 190
NCK1 = 126
NCHUNK = NCK0 + NCK1
EP = NS * NCHUNK * CH
NCHT = NS * NCHUNK
NP2 = 10112
RPT = NP2 // NS
CW = SH // 2
RW = RV // 2
EREC = 19 * CH

_sc_mesh = plsc.VectorSubcoreMesh(core_axis_name="c", subcore_axis_name="s")


def _unpack2(w):
    lo = plsc.bitcast(jnp.left_shift(w, 16), jnp.float32)
    hi = plsc.bitcast(jnp.bitwise_and(w, -65536), jnp.float32)
    return lo, hi


def _compute_chunk(eb, hb, combo_v):
    @plsc.parallel_loop(0, CH // 8)
    def _(bi):
        for i in range(8):
            e = bi * 8 + i
            rv = eb[pl.ds(pl.multiple_of(3 * CH + e * 16, 16), 16)]
            rlo, rhi = _unpack2(rv)
            for k in range(4):
                hw = plsc.bitcast(hb[e, pl.ds(16 * k, 16)], jnp.int32)
                cw = combo_v[e, pl.ds(16 * k, 16)]
                hlo, hhi = _unpack2(hw)
                clo, chi = _unpack2(cw)
                if k == 3:
                    hlo = hlo + rlo
                    hhi = hhi + rhi
                hb[e, pl.ds(16 * k, 16)] = jnp.maximum(hlo + clo, 0.0)
                hb[e, pl.ds(64 + 16 * k, 16)] = jnp.maximum(hhi + chi, 0.0)


def _msg_body(h, edata, comboP, zrows, out,
              shared, scombo, ebufA, ebufB, dbuf, hbufA, hbufB, cbufA, cbufB,
              gsemA, gsemB, lsemA, lsemB, csemA, csemB, sem_s):
    c = lax.axis_index("c")
    s = lax.axis_index("s")

    @pl.when(s == 0)
    def _():
        pltpu.sync_copy(comboP, scombo)

    pltpu.sync_copy(zrows, shared.at[pl.ds(s * RPT, RPT)])
    plsc.subcore_barrier()

    is0 = c == 0
    ci0 = jnp.where(is0, s * NCK0, NS * NCK0 + s * NCK1)
    nloc = jnp.where(is0, NCK0, NCK1)
    ebufs = (ebufA, ebufB)
    hbufs = (hbufA, hbufB)
    cbufs = (cbufA, cbufB)
    gsems = (gsemA, gsemB)
    lsems = (lsemA, lsemB)
    csems = (csemA, csemB)

    def src_idx(eb):
        return eb.at[pl.ds(0, CH)]

    def cid_idx(eb):
        return eb.at[pl.ds(2 * CH, CH)]

    pltpu.sync_copy(edata.at[pl.ds(ci0 * EREC, EREC)], ebufA)
    pltpu.async_copy(edata.at[pl.ds((ci0 + 1) * EREC, EREC)], ebufB, lsemB)
    pltpu.async_copy(h.at[src_idx(ebufA)], hbufA, gsemA)
    pltpu.async_copy(scombo.at[cid_idx(ebufA)], cbufA, csemA)

    def pair_body(p, carry):
        for b in range(2):
            eb, hb, cb = ebufs[b], hbufs[b], cbufs[b]
            ebn, hbn, cbn = ebufs[1 - b], hbufs[1 - b], cbufs[1 - b]
            j = p * 2 + b

            def wait_scatter(hbn=hbn, b=b):
                pltpu.make_async_copy(
                    hbn, shared.at[dbuf.at[1 - b]], sem_s).wait()
            if b == 0:
                @pl.when(p > 0)
                def _(wait_scatter=wait_scatter):
                    wait_scatter()
            else:
                wait_scatter()
            pltpu.make_async_copy(edata.at[pl.ds(0, EREC)], ebn,
                                  lsems[1 - b]).wait()
            pltpu.async_copy(h.at[src_idx(ebn)], hbn, gsems[1 - b])
            pltpu.async_copy(scombo.at[cid_idx(ebn)], cbn, csems[1 - b])
            pltpu.make_async_copy(h.at[src_idx(eb)], hb, gsems[b]).wait()
            pltpu.make_async_copy(scombo.at[cid_idx(eb)], cb, csems[b]).wait()
            for k in range(NG):
                dbuf[b, pl.ds(k * 16, 16)] = eb[pl.ds(CH + k * 16, 16)]
            _compute_chunk(eb, hb, cb)
            ci2 = ci0 + jnp.minimum(j + 2, nloc - 1)
            pltpu.async_copy(edata.at[pl.ds(ci2 * EREC, EREC)], eb, lsems[b])
            pltpu.async_copy(hb, shared.at[dbuf.at[b]], sem_s, add=True)
        return carry

    lax.fori_loop(0, nloc // 2, pair_body, 0)
    pltpu.make_async_copy(hbufB, shared.at[dbuf.at[1]], sem_s).wait()
    pltpu.make_async_copy(h.at[src_idx(ebufA)], hbufA, gsemA).wait()
    pltpu.make_async_copy(scombo.at[cid_idx(ebufA)], cbufA, csemA).wait()
    pltpu.make_async_copy(edata.at[pl.ds(0, EREC)], ebufB, lsemB).wait()
    plsc.subcore_barrier()

    rb = s * RPT
    pltpu.sync_copy(shared.at[pl.ds(rb, RPT)], out.at[c, pl.ds(rb, RPT)])


_msg_kernel = functools.partial(
    pl.kernel,
    mesh=_sc_mesh,
    compiler_params=pltpu.CompilerParams(needs_layout_passes=False),
    out_type=jax.ShapeDtypeStruct((NC, NP2, H), jnp.float32),
    scratch_types=[
        pltpu.VMEM_SHARED((NP2, H), jnp.float32),
        pltpu.VMEM_SHARED((512, 128), jnp.int32),
        pltpu.VMEM((EREC,), jnp.int32),
        pltpu.VMEM((EREC,), jnp.int32),
        pltpu.VMEM((2, CH), jnp.int32),
        pltpu.VMEM((CH, H), jnp.float32),
        pltpu.VMEM((CH, H), jnp.float32),
        pltpu.VMEM((CH, 128), jnp.int32),
        pltpu.VMEM((CH, 128), jnp.int32),
        pltpu.SemaphoreType.DMA,
        pltpu.SemaphoreType.DMA,
        pltpu.SemaphoreType.DMA,
        pltpu.SemaphoreType.DMA,
        pltpu.SemaphoreType.DMA,
        pltpu.SemaphoreType.DMA,
        pltpu.SemaphoreType.DMA,
    ],
)(_msg_body)


def _tc_pack(v):
    b = lax.bitcast_convert_type(v.astype(jnp.bfloat16), jnp.uint16)
    lo = b[:, :64].astype(jnp.uint32)
    hi = b[:, 64:].astype(jnp.uint32)
    p = lax.bitcast_convert_type(lo | (hi << 16), jnp.float32)
    return jnp.concatenate([p, p], axis=1)


def _enc_body(nf_ref, emb_ref, rx_ref, out_ref, pk_ref):
    acc = rx_ref[...]
    nf = nf_ref[...]
    for i in range(9):
        col = nf[:, i][:, None]
        oh = (col == lax.broadcasted_iota(jnp.int32, (1, 64), 1)).astype(jnp.float32)
        acc = acc + jnp.dot(oh, emb_ref[i], preferred_element_type=jnp.float32)
    out_ref[...] = acc
    pk_ref[...] = _tc_pack(acc)


def _mlp_body(h_ref, a0_ref, a1_ref, w1_ref, b1_ref, g1_ref, be1_ref,
              w2_ref, b2_ref, go_ref, bo_ref, eps_ref, out_ref, pk_ref, *,
              relu_out):
    x = h_ref[...]
    t = (1.0 + eps_ref[0, 0]) * x + a0_ref[0] + a1_ref[0]
    u = jnp.dot(t, w1_ref[...], preferred_element_type=jnp.float32) + b1_ref[...]
    u = jnp.maximum(g1_ref[...] * u + be1_ref[...], 0.0)
    v = jnp.dot(u, w2_ref[...], preferred_element_type=jnp.float32) + b2_ref[...]
    v = go_ref[...] * v + bo_ref[...]
    if relu_out:
        v = jnp.maximum(v, 0.0)
    out_ref[...] = v
    pk_ref[...] = _tc_pack(v)


_R = 400
_G = N // _R

_row_spec = pl.BlockSpec((_R, H), lambda i: (i, 0))
_pk_spec = pl.BlockSpec((_R, H), lambda i: (i, 0))
_vec_spec = pl.BlockSpec((1, H), lambda i: (0, 0))
_mat_spec = pl.BlockSpec((H, H), lambda i: (0, 0))
_out_shapes = [jax.ShapeDtypeStruct((N, H), jnp.float32),
               jax.ShapeDtypeStruct((N, H), jnp.float32)]


def _make_mlp(relu_out):
    return pl.pallas_call(
        functools.partial(_mlp_body, relu_out=relu_out),
        grid=(_G,),
        in_specs=[_row_spec,
                  pl.BlockSpec((1, _R, H), lambda i: (0, i, 0)),
                  pl.BlockSpec((1, _R, H), lambda i: (1, i, 0)),
                  _mat_spec, _vec_spec,
                  _vec_spec, _vec_spec, _mat_spec, _vec_spec, _vec_spec,
                  _vec_spec, pl.BlockSpec((1, 1), lambda i: (0, 0))],
        out_specs=[_row_spec, _pk_spec],
        out_shape=_out_shapes,
    )


_mlp_relu = _make_mlp(True)
_mlp_last = _make_mlp(False)

_encoder = pl.pallas_call(
    _enc_body,
    grid=(_G,),
    in_specs=[pl.BlockSpec((_R, 128), lambda i: (i, 0)),
              pl.BlockSpec((9, 64, H), lambda i: (0, 0, 0)),
              _row_spec],
    out_specs=[_row_spec, _pk_spec],
    out_shape=_out_shapes,
)


def _pack_split(x, k):
    bits = lax.bitcast_convert_type(x.astype(jnp.bfloat16), jnp.uint16)
    bits = bits.astype(jnp.uint32)
    return (bits[..., :k] | (bits[..., k:2 * k] << 16)).astype(jnp.int32)


_PERM = np.concatenate([np.arange(0, 59), np.arange(118, 123),
                        np.arange(59, 118), np.arange(123, 128)])


def kernel(rand_x, rand_edge, edge_index, node_feat, edge_attr, atom_emb,
           bond_emb, W1, b1, g1, be1, W2, b2, go, bo, eps):
    f32 = jnp.float32
    src = edge_index[0]
    dst = edge_index[1]
    cid = edge_attr[:, 0] * 64 + edge_attr[:, 1] * 8 + edge_attr[:, 2]
    pad = EP - E
    srcp = jnp.concatenate([src, jnp.zeros((pad,), jnp.int32)])
    dstp = jnp.concatenate([dst, jnp.full((pad,), N, jnp.int32)])
    cidp = jnp.concatenate([cid, jnp.zeros((pad,), jnp.int32)])
    randp = jnp.concatenate([_pack_split(rand_edge, RW),
                             jnp.zeros((pad, RW), jnp.int32)], axis=0)
    randp = jnp.concatenate([jnp.zeros((EP, 16 - RW), jnp.int32), randp],
                            axis=1)
    base3 = jnp.stack([srcp, dstp, cidp]).reshape(3, NCHT, CH)
    base3 = base3.transpose(1, 0, 2).reshape(NCHT, 3 * CH)
    randb = randp.reshape(NCHT, CH * 16)
    edata = jnp.concatenate([base3, randb], axis=1).reshape(-1)

    ii = jnp.arange(512)
    combo = (bond_emb[:, 0, ii // 64, :] + bond_emb[:, 1, (ii // 8) % 8, :]
             + bond_emb[:, 2, ii % 8, :])
    comboP = jnp.concatenate([_pack_split(combo, CW),
                              jnp.zeros((L, 512, 128 - CW), jnp.int32)],
                             axis=2)
    zrows = jnp.zeros((RPT, H), f32)

    nz = jnp.zeros((N, 5), f32)
    ez = jnp.zeros((9, 64, 5), f32)
    nfp = jnp.pad(node_feat, ((0, 0), (0, 128 - 9)))
    rxp = jnp.concatenate([jnp.zeros((N, CW), f32), rand_x[:, :5],
                           jnp.zeros((N, CW), f32), rand_x[:, 5:]], axis=1)
    embp = jnp.concatenate([atom_emb[:, :, :CW], ez,
                            atom_emb[:, :, CW:SH], ez], axis=2)

    h, hp = _encoder(nfp, embp, rxp)
    for l in range(L):
        sc_out = _msg_kernel(hp, edata, comboP[l], zrows)
        last = l == L - 1
        mlp = _mlp_last if last else _mlp_relu
        w2l = W2[l] if last else W2[l][:, _PERM]
        b2l = b2[l] if last else b2[l][_PERM]
        gol = go[l] if last else go[l][_PERM]
        bol = bo[l] if last else bo[l][_PERM]
        h, hp = mlp(h, sc_out, sc_out, W1[l][_PERM, :], b1[l][None],
                    g1[l][None], be1[l][None], w2l, b2l[None], gol[None],
                    bol[None], eps[l].reshape(1, 1))
    return h
